# baseline (speedup 1.0000x reference)
import jax
import jax.numpy as jnp
from jax.experimental import pallas as pl

B = 8
S = 100
N = 64
D = 256
K = 10
IT = 15
M = B * S
CH = 8
NG = 5
G = M // (CH * NG)
R = CH * N
KL = 128


def _kmeans_block(x_ref, cid_ref, ctr_ref, loss_ref):
    i = pl.program_id(0)

    lane_g = jax.lax.broadcasted_iota(jnp.int32, (R, KL), 1)
    row_r = jax.lax.broadcasted_iota(jnp.int32, (R, KL), 0)
    valid = (lane_g < CH * K) & ((lane_g // K) == (row_r // N))
    neg = jnp.float32(-1e30)
    sub_g = jax.lax.broadcasted_iota(jnp.int32, (KL, R), 0)

    X2s, Xas, x_sqs, c0s = [], [], [], []
    for g in range(NG):
        Xg = x_ref[g * R:(g + 1) * R, :]
        X2s.append(Xg + Xg)
        Xas.append(jnp.concatenate(
            [Xg, jnp.ones((R, 1), jnp.float32)], axis=1))
        x_sqs.append(jnp.sum(Xg * Xg, axis=1, keepdims=True))
        c0s.append(jnp.concatenate(
            [Xg[ci * N:ci * N + K] for ci in range(CH)]
            + [jnp.zeros((KL - CH * K, D), jnp.float32)], axis=0))

    def assign(g, centers):
        c_sq = jnp.sum(centers * centers, axis=1)
        dot2 = jax.lax.dot_general(X2s[g], centers,
                                   (((1,), (1,)), ((), ())))
        sim = dot2 - x_sqs[g] - c_sq[None, :]
        sim = jnp.where(valid, sim, neg)
        cid = jnp.argmax(sim, axis=1)
        onehot_t = (sub_g == cid[None, :]).astype(jnp.float32)
        return sim, cid, onehot_t

    def step(g, centers):
        sim, cid, onehot_t = assign(g, centers)
        sums_aug = jax.lax.dot_general(
            onehot_t, Xas[g], (((1,), (0,)), ((), ())))
        counts = sums_aug[:, D:D + 1]
        new_c = sums_aug[:, :D] / jnp.maximum(counts, 1.0)
        new_c = jnp.where(counts > 0.0, new_c, centers)
        return new_c, sim, cid, counts

    def cond(carry):
        it, _, _, conv = carry
        return (it < IT) & jnp.logical_not(conv)

    def body(carry):
        it, cs, prev_cids, _ = carry
        new_cs, cids = [], []
        ndiff = jnp.float32(0.0)
        for g in range(NG):
            new_c, _, cid, _ = step(g, cs[g])
            new_cs.append(new_c)
            cids.append(cid)
            ndiff += jnp.sum((cid != prev_cids[g]).astype(jnp.float32))
        return it + 1, tuple(new_cs), tuple(cids), ndiff == 0.0

    prev0 = tuple(jnp.full((R,), -1, jnp.int32) for _ in range(NG))
    _, centers_f, _, _ = jax.lax.while_loop(
        cond, body, (jnp.int32(0), tuple(c0s), prev0, jnp.bool_(False)))

    kml = jnp.float32(0.0)
    uni = jnp.float32(0.0)
    g1 = jax.lax.broadcasted_iota(jnp.int32, (KL, 1), 0)
    valid_g = g1 < CH * K
    for g in range(NG):
        sim, cid, onehot_t = assign(g, centers_f[g])
        counts = jnp.sum(onehot_t, axis=1, keepdims=True)
        cid_ref[g * R:(g + 1) * R, :] = (cid % K).astype(jnp.int32)[:, None]
        ctr_ref[g * CH * K:(g + 1) * CH * K, :] = centers_f[g][:CH * K, :]

        best = jnp.max(sim, axis=1, keepdims=True)
        mds = jnp.maximum(-best, 0.0)
        kml += jnp.sum(mds) / float(M * N)

        frac = counts / float(N)
        uni += jnp.sum(
            jnp.where(valid_g, (frac - 1.0 / K) ** 2, 0.0)) / float(M * K)

    p_r = jax.lax.broadcasted_iota(jnp.int32, (8, 128), 0)
    p_l = jax.lax.broadcasted_iota(jnp.int32, (8, 128), 1)
    acc = (kml * ((p_r == 0) & (p_l == 0)).astype(jnp.float32)
           + uni * ((p_r == 0) & (p_l == 1)).astype(jnp.float32))

    @pl.when(i == 0)
    def _():
        loss_ref[...] = jnp.zeros((8, 128), jnp.float32)

    loss_ref[...] += acc


def kernel(feature):
    x = feature.reshape(M * N, D)

    cid_flat, ctr_flat, loss = pl.pallas_call(
        _kmeans_block,
        grid=(G,),
        in_specs=[pl.BlockSpec((NG * R, D), lambda i: (i, 0))],
        out_specs=[
            pl.BlockSpec((NG * R, 1), lambda i: (i, 0)),
            pl.BlockSpec((NG * CH * K, D), lambda i: (i, 0)),
            pl.BlockSpec((8, 128), lambda i: (0, 0)),
        ],
        out_shape=[
            jax.ShapeDtypeStruct((M * N, 1), jnp.int32),
            jax.ShapeDtypeStruct((M * K, D), jnp.float32),
            jax.ShapeDtypeStruct((8, 128), jnp.float32),
        ],
    )(x)

    cid = cid_flat.reshape(B, S, N)
    centers = ctr_flat.reshape(B, S, K, D)
    return (cid, centers, loss[0, 0], loss[0, 1])


# speedup vs baseline: 1.0527x; 1.0527x over previous
import jax
import jax.numpy as jnp
from jax.experimental import pallas as pl

B = 8
S = 100
N = 64
D = 256
K = 10
IT = 15
M = B * S
CH = 8
NG = 5
G = M // (CH * NG)
R = CH * N
KL = 128


def _kmeans_block(x_ref, cid_ref, ctr_ref, loss_ref):
    i = pl.program_id(0)

    lane_g = jax.lax.broadcasted_iota(jnp.int32, (R, KL), 1)
    row_r = jax.lax.broadcasted_iota(jnp.int32, (R, KL), 0)
    valid = (lane_g < CH * K) & ((lane_g // K) == (row_r // N))
    inf_bias = jnp.where(valid, 0.0, jnp.inf).astype(jnp.float32)

    X2s, Xts, x_sqs, c0s = [], [], [], []
    for g in range(NG):
        Xg = x_ref[g * R:(g + 1) * R, :]
        X2s.append(Xg + Xg)
        Xts.append(jnp.concatenate(
            [jnp.swapaxes(Xg, 0, 1), jnp.ones((1, R), jnp.float32)],
            axis=0))
        x_sq = jnp.sum(Xg * Xg, axis=1, keepdims=True)
        x_sqs.append(x_sq + inf_bias)
        c0 = jnp.concatenate(
            [Xg[ci * N:ci * N + K] for ci in range(CH)]
            + [jnp.zeros((KL - CH * K, D), jnp.float32)], axis=0)
        c0s.append(jnp.swapaxes(c0, 0, 1))

    def assign(g, ct):
        c_sq = jnp.sum(ct * ct, axis=0, keepdims=True)
        dot2 = jax.lax.dot_general(X2s[g], ct,
                                   (((1,), (0,)), ((), ())))
        sim = dot2 - x_sqs[g] - c_sq
        cid = jnp.argmax(sim, axis=1, keepdims=True)
        onehot = (lane_g == cid).astype(jnp.float32)
        return sim, cid, onehot

    def step(g, ct):
        sim, cid, onehot = assign(g, ct)
        sums_t = jax.lax.dot_general(
            Xts[g], onehot, (((1,), (0,)), ((), ())))
        counts = sums_t[D:D + 1, :]
        new_ct = sums_t[:D, :] / jnp.maximum(counts, 1.0)
        new_ct = jnp.where(counts > 0.0, new_ct, ct)
        return new_ct, sim, cid, counts

    def cond(carry):
        it, _, _, conv = carry
        return (it < IT) & jnp.logical_not(conv)

    def body(carry):
        it, cs, prev_cids, _ = carry
        new_cs, cids = [], []
        ndiff = jnp.float32(0.0)
        for g in range(NG):
            new_ct, _, cid, _ = step(g, cs[g])
            new_cs.append(new_ct)
            cids.append(cid)
            ndiff += jnp.sum((cid != prev_cids[g]).astype(jnp.float32))
        return it + 1, tuple(new_cs), tuple(cids), ndiff == 0.0

    prev0 = tuple(jnp.full((R, 1), -1, jnp.int32) for _ in range(NG))
    _, centers_f, _, _ = jax.lax.while_loop(
        cond, body, (jnp.int32(0), tuple(c0s), prev0, jnp.bool_(False)))

    kml = jnp.float32(0.0)
    uni = jnp.float32(0.0)
    lane_1 = jax.lax.broadcasted_iota(jnp.int32, (1, KL), 1)
    valid_l = lane_1 < CH * K
    for g in range(NG):
        sim, cid, onehot = assign(g, centers_f[g])
        counts = jnp.sum(onehot, axis=0, keepdims=True)
        cid_ref[g * R:(g + 1) * R, :] = (cid % K).astype(jnp.int32)
        ctr_ref[g * CH * K:(g + 1) * CH * K, :] = (
            jnp.swapaxes(centers_f[g], 0, 1)[:CH * K, :])

        best = jnp.max(sim, axis=1, keepdims=True)
        mds = jnp.maximum(-best, 0.0)
        kml += jnp.sum(mds) / float(M * N)

        frac = counts / float(N)
        uni += jnp.sum(
            jnp.where(valid_l, (frac - 1.0 / K) ** 2, 0.0)) / float(M * K)

    p_r = jax.lax.broadcasted_iota(jnp.int32, (8, 128), 0)
    p_l = jax.lax.broadcasted_iota(jnp.int32, (8, 128), 1)
    acc = (kml * ((p_r == 0) & (p_l == 0)).astype(jnp.float32)
           + uni * ((p_r == 0) & (p_l == 1)).astype(jnp.float32))

    @pl.when(i == 0)
    def _():
        loss_ref[...] = jnp.zeros((8, 128), jnp.float32)

    loss_ref[...] += acc


def kernel(feature):
    x = feature.reshape(M * N, D)

    cid_flat, ctr_flat, loss = pl.pallas_call(
        _kmeans_block,
        grid=(G,),
        in_specs=[pl.BlockSpec((NG * R, D), lambda i: (i, 0))],
        out_specs=[
            pl.BlockSpec((NG * R, 1), lambda i: (i, 0)),
            pl.BlockSpec((NG * CH * K, D), lambda i: (i, 0)),
            pl.BlockSpec((8, 128), lambda i: (0, 0)),
        ],
        out_shape=[
            jax.ShapeDtypeStruct((M * N, 1), jnp.int32),
            jax.ShapeDtypeStruct((M * K, D), jnp.float32),
            jax.ShapeDtypeStruct((8, 128), jnp.float32),
        ],
    )(x)

    cid = cid_flat.reshape(B, S, N)
    centers = ctr_flat.reshape(B, S, K, D)
    return (cid, centers, loss[0, 0], loss[0, 1])
